# Initial kernel scaffold; baseline (speedup 1.0000x reference)
import jax
import jax.numpy as jnp
from jax.experimental import pallas as pl


def kernel(tokens, lens, params):
    raise NotImplementedError("write your pallas kernel here")


# speedup vs baseline: 1.8631x; 1.8631x over previous
import jax
import jax.numpy as jnp
from jax.experimental import pallas as pl
from jax.experimental.pallas import tpu as pltpu

HEADS = 8
DIM = 512
SEQ = 512
NEG = -1e9


def _bdot(a, b, dn):
    return jax.lax.dot_general(a.astype(jnp.bfloat16), b.astype(jnp.bfloat16),
                               dn, preferred_element_type=jnp.float32)


def _dot(a, b):
    return _bdot(a, b, (((1,), (0,)), ((), ())))


def _dot_nt(a, b):
    return _bdot(a, b, (((1,), (1,)), ((), ())))


def _xsum(x):
    w = x.shape[1]
    y = x[:, 0:128]
    for i in range(1, w // 128):
        y = y + x[:, 128 * i:128 * (i + 1)]
    acc = y[:, 0:8]
    for k in range(1, 16):
        acc = acc + y[:, 8 * k:8 * (k + 1)]
    t = acc[:, 0:4] + acc[:, 4:8]
    t = t[:, 0:2] + t[:, 2:4]
    return t[:, 0:1] + t[:, 1:2]


def _nsum(x):
    return jnp.sum(x, axis=-1, keepdims=True)


def _split3(v):
    h = v.astype(jnp.bfloat16)
    r = v - h.astype(jnp.float32)
    m = r.astype(jnp.bfloat16)
    l = (r - m.astype(jnp.float32)).astype(jnp.bfloat16)
    return h, m, l


def _exact_onehot_dot(G, data, dn):
    Gb = G.astype(jnp.bfloat16)
    h, m, l = _split3(data)
    f32 = jnp.float32
    d = jax.lax.dot_general(Gb, h, dn, preferred_element_type=f32)
    d = d + jax.lax.dot_general(Gb, m, dn, preferred_element_type=f32)
    return d + jax.lax.dot_general(Gb, l, dn, preferred_element_type=f32)


def _ln(x, g, sumfn):
    mu = sumfn(x) * (1.0 / DIM)
    xc = x - mu
    var = sumfn(xc * xc) * (1.0 / DIM)
    return g * xc / jnp.sqrt(var + 1e-5)


def _mha(x, Wq, Wk, Wv, Wo, kmask_row, sumfn):
    S = x.shape[0]
    q = _dot(x, Wq)
    k = _dot(x, Wk)
    v = _dot(x, Wv)
    ii = jax.lax.broadcasted_iota(jnp.int32, (S, S), 0)
    jj = jax.lax.broadcasted_iota(jnp.int32, (S, S), 1)
    mask = ii >= jj
    if kmask_row is not None:
        mask = jnp.logical_and(mask, kmask_row)
    hd = DIM // HEADS
    outs = []
    for h in range(HEADS):
        sl = slice(h * hd, (h + 1) * hd)
        s = _dot_nt(q[:, sl], k[:, sl]) / 8.0
        s = jnp.where(mask, s, NEG)
        m = jnp.max(s, axis=-1, keepdims=True)
        e = jnp.exp(s - m)
        a = e / sumfn(e)
        outs.append(_dot(a, v[:, sl]))
    o = jnp.concatenate(outs, axis=-1)
    return _dot(o, Wo)


def _block(x, g1, Wq, Wk, Wv, Wo, g2, W1, W2, sumfn, kmask_row=None):
    x = x + _mha(_ln(x, g1, sumfn), Wq, Wk, Wv, Wo, kmask_row, sumfn)
    h = _ln(x, g2, sumfn)
    x = x + _dot(jax.nn.gelu(_dot(h, W1)), W2)
    return x


def _k1_body(tokens_ref, lens_ref,
             eg1, eWq, eWk, eWv, eWo, eg2, eW1, eW2, cWq, cWk,
             enc_ref, down_ref, aux_ref, loss_ref):
    b = pl.program_id(0)
    x = tokens_ref[0]
    L = lens_ref[b]

    row = jax.lax.broadcasted_iota(jnp.int32, (SEQ, 1), 0)
    col = jax.lax.broadcasted_iota(jnp.int32, (1, SEQ), 1)
    valid_col = (row < L).astype(jnp.float32)

    encoded = _block(x, eg1[...], eWq[...], eWk[...], eWv[...], eWo[...],
                     eg2[...], eW1[...], eW2[...], _xsum)
    enc_ref[0] = encoded

    q = _dot(encoded, cWq[...])
    k = _dot(encoded, cWk[...])
    qn = q / (jnp.sqrt(_xsum(q * q)) + 1e-6)
    kn = k / (jnp.sqrt(_xsum(k * k)) + 1e-6)
    kn_next = pltpu.roll(kn, SEQ - 1, 0)
    cos = _xsum(qn * kn_next)
    p_at_next = jnp.clip((1.0 - cos) * 0.5, 0.0, 1.0)
    p_prev = pltpu.roll(p_at_next, 1, 0)
    pvec = jnp.where(row == 0, 1.0, p_prev)
    bf = jnp.logical_and(pvec >= 0.5, row < L).astype(jnp.float32)

    ii = jax.lax.broadcasted_iota(jnp.int32, (SEQ, SEQ), 0)
    jj = jax.lax.broadcasted_iota(jnp.int32, (SEQ, SEQ), 1)
    tri = (jj <= ii).astype(jnp.float32)
    cum = _bdot(tri, bf, (((1,), (0,)), ((), ())))
    cid = jnp.clip(cum - 1.0, 0.0, float(SEQ - 1))
    nchunks = jnp.sum(bf)
    cvalid_col = (row.astype(jnp.float32) < nchunks).astype(jnp.float32)
    G = (cid == col.astype(jnp.float32)).astype(jnp.float32)
    down = _exact_onehot_dot(G * bf, encoded, (((0,), (0,)), ((), ())))
    down_ref[0] = down * cvalid_col

    lane = jax.lax.broadcasted_iota(jnp.int32, (SEQ, 128), 1)
    aux_ref[0] = (jnp.where(lane == 0, cid, 0.0)
                  + jnp.where(lane == 1, nchunks, 0.0))

    lane1 = jax.lax.broadcasted_iota(jnp.int32, (1, 128), 1)
    vsum = jnp.sum(valid_col)
    psum = jnp.sum(pvec * valid_col)
    part = (jnp.where(lane1 == 1, vsum, 0.0)
            + jnp.where(lane1 == 2, nchunks, 0.0)
            + jnp.where(lane1 == 3, psum, 0.0))

    @pl.when(b == 0)
    def _():
        loss_ref[...] = part

    @pl.when(b > 0)
    def _():
        loss_ref[...] = loss_ref[...] + part

    @pl.when(b == pl.num_programs(0) - 1)
    def _():
        acc = loss_ref[...]
        vs = jnp.sum(jnp.where(lane1 == 1, acc, 0.0))
        bs = jnp.sum(jnp.where(lane1 == 2, acc, 0.0))
        ps = jnp.sum(jnp.where(lane1 == 3, acc, 0.0))
        F = bs / vs
        Gv = ps / vs
        loss = 1.5 * (2.0 * F * Gv + (1.0 - F) * (1.0 - Gv))
        loss_ref[...] = jnp.where(lane1 == 0, loss, acc)


def _k2_body(down_ref, aux_ref,
             ng1, nWq, nWk, nWv, nWo, ng2, nW1, nW2,
             inner_ref):
    down = down_ref[0]
    aux = aux_ref[0]
    lane = jax.lax.broadcasted_iota(jnp.int32, (SEQ, 128), 1)
    nchunks = jnp.max(jnp.where(lane == 1, aux, 0.0))
    row = jax.lax.broadcasted_iota(jnp.int32, (SEQ, 1), 0)
    col = jax.lax.broadcasted_iota(jnp.int32, (1, SEQ), 1)
    cvalid_col = (row.astype(jnp.float32) < nchunks).astype(jnp.float32)
    cvalid_row = col.astype(jnp.float32) < nchunks
    inner = _block(down, ng1[...], nWq[...], nWk[...], nWv[...], nWo[...],
                   ng2[...], nW1[...], nW2[...], _nsum, kmask_row=cvalid_row)
    inner_ref[0] = inner * cvalid_col


def _k3_body(inner_ref, enc_ref, aux_ref, lens_ref, cWres,
             dg1, dWq, dWk, dWv, dWo, dg2, dW1, dW2,
             out_ref):
    b = pl.program_id(0)
    inner = inner_ref[0]
    encoded = enc_ref[0]
    aux = aux_ref[0]
    L = lens_ref[b]
    lane = jax.lax.broadcasted_iota(jnp.int32, (SEQ, 128), 1)
    cid = jnp.sum(jnp.where(lane == 0, aux, 0.0), axis=-1, keepdims=True)
    row = jax.lax.broadcasted_iota(jnp.int32, (SEQ, 1), 0)
    col = jax.lax.broadcasted_iota(jnp.int32, (1, SEQ), 1)
    valid_col = (row < L).astype(jnp.float32)
    G = (cid == col.astype(jnp.float32)).astype(jnp.float32)
    gathered = _exact_onehot_dot(G, inner, (((1,), (0,)), ((), ())))
    ups = (gathered + _dot(encoded, cWres[...])) * valid_col
    out = _block(ups, dg1[...], dWq[...], dWk[...], dWv[...], dWo[...],
                 dg2[...], dW1[...], dW2[...], _nsum) * valid_col
    out_ref[0] = out


def _wspec(w):
    return pl.BlockSpec(w.shape, lambda b: (0,) * w.ndim)


def _bspec(S, D):
    return pl.BlockSpec((1, S, D), lambda b: (b, 0, 0))


def _blkw(p):
    D = DIM
    return [p['g1'].reshape(1, D), p['Wq'], p['Wk'], p['Wv'], p['Wo'],
            p['g2'].reshape(1, D), p['W1'], p['W2']]


def _run(tokens, lens, params, interpret=False):
    B, S, D = tokens.shape
    lens32 = lens.astype(jnp.int32)
    f32 = jnp.float32

    w1 = _blkw(params['enc']) + [params['chk']['Wq'], params['chk']['Wk']]
    encoded, down, aux, loss = pl.pallas_call(
        _k1_body,
        grid=(B,),
        in_specs=[_bspec(S, D), pl.BlockSpec(memory_space=pltpu.SMEM)]
                 + [_wspec(w) for w in w1],
        out_specs=[_bspec(S, D), _bspec(S, D), _bspec(S, 128),
                   pl.BlockSpec((1, 128), lambda b: (0, 0))],
        out_shape=[jax.ShapeDtypeStruct((B, S, D), f32),
                   jax.ShapeDtypeStruct((B, S, D), f32),
                   jax.ShapeDtypeStruct((B, S, 128), f32),
                   jax.ShapeDtypeStruct((1, 128), f32)],
        interpret=interpret,
    )(tokens, lens32, *w1)

    w2 = _blkw(params['net'])
    inner = pl.pallas_call(
        _k2_body,
        grid=(B,),
        in_specs=[_bspec(S, D), _bspec(S, 128)] + [_wspec(w) for w in w2],
        out_specs=_bspec(S, D),
        out_shape=jax.ShapeDtypeStruct((B, S, D), f32),
        interpret=interpret,
    )(down, aux, *w2)

    w3 = [params['chk']['Wres']] + _blkw(params['dec'])
    out = pl.pallas_call(
        _k3_body,
        grid=(B,),
        in_specs=[_bspec(S, D), _bspec(S, D), _bspec(S, 128),
                  pl.BlockSpec(memory_space=pltpu.SMEM)]
                 + [_wspec(w) for w in w3],
        out_specs=_bspec(S, D),
        out_shape=jax.ShapeDtypeStruct((B, S, D), f32),
        interpret=interpret,
    )(inner, encoded, aux, lens32, *w3)

    return out, loss[0, 0]


def kernel(tokens, lens, params):
    return _run(tokens, lens, params)
